# baseline (speedup 1.0000x reference)
import jax
import jax.numpy as jnp
from jax import lax
from jax.experimental import pallas as pl
from jax.experimental.pallas import tpu as pltpu
from jax.experimental.pallas import tpu_sc as plsc

_N_BINS = 256
_N_ELEM = 2 * 3 * 224 * 224
_NC = 2
_NS = 16
_NW = _NC * _NS
_ROWS = 1344
_W = 224
_RPT = 42
_WIN = 56
_COLS = _N_BINS
_HWORDS = 16 * _COLS + 16
_SCALE = 1.0 / (float(_N_ELEM) ** 2 * float(_N_BINS))


def _sc_body(x_hbm, y_hbm, outx_hbm, outy_hbm, bufx, bufy, hx, hy,
             semx, semy):
    wid = lax.axis_index("s") * _NC + lax.axis_index("c")

    start = wid * _RPT
    base = jnp.minimum((start // 8) * 8, _ROWS - _WIN)
    cpx = pltpu.async_copy(x_hbm.at[pl.ds(base, _WIN), :], bufx, semx)
    cpy = pltpu.async_copy(y_hbm.at[pl.ds(base, _WIN), :], bufy, semy)

    zero = jnp.zeros((16,), jnp.float32)

    @plsc.parallel_loop(0, _HWORDS, step=16, unroll=8)
    def _zero(o):
        hx[pl.ds(o, 16)] = zero
        hy[pl.ds(o, 16)] = zero

    lane_off = lax.iota(jnp.int32, 16) * _COLS
    one = jnp.float32(1.0)
    r0 = start - base

    def _accum_into(buf, h):
        @plsc.parallel_loop(0, _RPT * (_W // 16), step=1, unroll=4,
                            carry=(jnp.int32(0), jnp.int32(0)))
        def _accum(i, rc):
            r, c = rc
            u = buf[r0 + r, pl.ds(c, 16)] * 255.0
            b = u.astype(jnp.int32)
            f = u - b.astype(jnp.float32)
            ix = lane_off + b
            plsc.addupdate_scatter(h, [ix], one - f)
            plsc.addupdate_scatter(h, [ix + 1], f)
            nc = c + 16
            wrap = nc >= _W
            return (jnp.where(wrap, r + 1, r), jnp.where(wrap, 0, nc))

    cpx.wait()
    _accum_into(bufx, hx)
    cpy.wait()
    _accum_into(bufy, hy)

    for off in (2048, 1024, 512, 256):
        @plsc.parallel_loop(0, off, step=16, unroll=2)
        def _fold(o, _off=off):
            hx[pl.ds(o, 16)] = hx[pl.ds(o, 16)] + hx[pl.ds(o + _off, 16)]
            hy[pl.ds(o, 16)] = hy[pl.ds(o, 16)] + hy[pl.ds(o + _off, 16)]

    pltpu.sync_copy(hx.at[pl.ds(0, _N_BINS)], outx_hbm.at[wid])
    pltpu.sync_copy(hy.at[pl.ds(0, _N_BINS)], outy_hbm.at[wid])


def _sc_hist(x2, y2):
    mesh = plsc.VectorSubcoreMesh(core_axis_name="c", subcore_axis_name="s")
    part = jax.ShapeDtypeStruct((_NW, _N_BINS), jnp.float32)
    f = pl.kernel(
        _sc_body,
        out_type=[part, part],
        mesh=mesh,
        compiler_params=pltpu.CompilerParams(
            needs_layout_passes=False, use_tc_tiling_on_sc=True),
        scratch_types=[
            pltpu.VMEM((_WIN, _W), jnp.float32),
            pltpu.VMEM((_WIN, _W), jnp.float32),
            pltpu.VMEM((_HWORDS,), jnp.float32),
            pltpu.VMEM((_HWORDS,), jnp.float32),
            pltpu.SemaphoreType.DMA,
            pltpu.SemaphoreType.DMA,
        ],
    )
    return f(x2, y2)


def _tc_loss_body(hx_ref, hy_ref, o_ref):
    d = jnp.sum(hx_ref[...] - hy_ref[...], axis=0, keepdims=True)
    s = jnp.sum(d * d) * _SCALE
    o_ref[...] = jnp.reshape(s, (1, 1))


def _tc_loss(hxp, hyp):
    return pl.pallas_call(
        _tc_loss_body,
        out_shape=jax.ShapeDtypeStruct((1, 1), jnp.float32),
    )(hxp, hyp)


def kernel(x, y):
    x2 = x.reshape(_ROWS, _W)
    y2 = y.reshape(_ROWS, _W)
    hxp, hyp = _sc_hist(x2, y2)
    return _tc_loss(hxp, hyp)[0, 0]


# speedup vs baseline: 1.0645x; 1.0645x over previous
import jax
import jax.numpy as jnp
from jax import lax
from jax.experimental import pallas as pl
from jax.experimental.pallas import tpu as pltpu
from jax.experimental.pallas import tpu_sc as plsc

_N_BINS = 256
_N_ELEM = 2 * 3 * 224 * 224
_NC = 2
_NS = 16
_NW = _NC * _NS
_ROWS = 1344
_W = 224
_RPT = 42
_WIN = 56
_COLS = _N_BINS
_HWORDS = 16 * _COLS + 16
_SCALE = 1.0 / (float(_N_ELEM) ** 2 * float(_N_BINS))


def _sc_body(x_hbm, y_hbm, outx_hbm, outy_hbm, bufx, bufy, hx, hy,
             semx, semy):
    wid = lax.axis_index("s") * _NC + lax.axis_index("c")

    start = wid * _RPT
    base = jnp.minimum((start // 8) * 8, _ROWS - _WIN)
    cpx = pltpu.async_copy(x_hbm.at[pl.ds(base, _WIN), :], bufx, semx)
    cpy = pltpu.async_copy(y_hbm.at[pl.ds(base, _WIN), :], bufy, semy)

    zero = jnp.zeros((16,), jnp.float32)

    @plsc.parallel_loop(0, _HWORDS, step=16, unroll=8)
    def _zero(o):
        hx[pl.ds(o, 16)] = zero
        hy[pl.ds(o, 16)] = zero

    lane_off = lax.iota(jnp.int32, 16) * _COLS
    one = jnp.float32(1.0)
    r0 = start - base

    cpx.wait()
    cpy.wait()

    @plsc.parallel_loop(0, _RPT * (_W // 16), step=1, unroll=4,
                        carry=(jnp.int32(0), jnp.int32(0)))
    def _accum(i, rc):
        r, c = rc
        ux = bufx[r0 + r, pl.ds(c, 16)] * 255.0
        bx = ux.astype(jnp.int32)
        fx = ux - bx.astype(jnp.float32)
        ix = lane_off + bx
        plsc.addupdate_scatter(hx, [ix], one - fx)
        plsc.addupdate_scatter(hx, [ix + 1], fx)
        uy = bufy[r0 + r, pl.ds(c, 16)] * 255.0
        by = uy.astype(jnp.int32)
        fy = uy - by.astype(jnp.float32)
        iy = lane_off + by
        plsc.addupdate_scatter(hy, [iy], one - fy)
        plsc.addupdate_scatter(hy, [iy + 1], fy)
        nc = c + 16
        wrap = nc >= _W
        return (jnp.where(wrap, r + 1, r), jnp.where(wrap, 0, nc))

    for off in (2048, 1024, 512, 256):
        @plsc.parallel_loop(0, off, step=16, unroll=2)
        def _fold(o, _off=off):
            hx[pl.ds(o, 16)] = hx[pl.ds(o, 16)] + hx[pl.ds(o + _off, 16)]
            hy[pl.ds(o, 16)] = hy[pl.ds(o, 16)] + hy[pl.ds(o + _off, 16)]

    pltpu.sync_copy(hx.at[pl.ds(0, _N_BINS)], outx_hbm.at[wid])
    pltpu.sync_copy(hy.at[pl.ds(0, _N_BINS)], outy_hbm.at[wid])


def _sc_hist(x2, y2):
    mesh = plsc.VectorSubcoreMesh(core_axis_name="c", subcore_axis_name="s")
    part = jax.ShapeDtypeStruct((_NW, _N_BINS), jnp.float32)
    f = pl.kernel(
        _sc_body,
        out_type=[part, part],
        mesh=mesh,
        compiler_params=pltpu.CompilerParams(
            needs_layout_passes=False, use_tc_tiling_on_sc=True),
        scratch_types=[
            pltpu.VMEM((_WIN, _W), jnp.float32),
            pltpu.VMEM((_WIN, _W), jnp.float32),
            pltpu.VMEM((_HWORDS,), jnp.float32),
            pltpu.VMEM((_HWORDS,), jnp.float32),
            pltpu.SemaphoreType.DMA,
            pltpu.SemaphoreType.DMA,
        ],
    )
    return f(x2, y2)


def _tc_loss_body(hx_ref, hy_ref, o_ref):
    d = jnp.sum(hx_ref[...] - hy_ref[...], axis=0, keepdims=True)
    s = jnp.sum(d * d) * _SCALE
    o_ref[...] = jnp.reshape(s, (1, 1))


def _tc_loss(hxp, hyp):
    return pl.pallas_call(
        _tc_loss_body,
        out_shape=jax.ShapeDtypeStruct((1, 1), jnp.float32),
    )(hxp, hyp)


def kernel(x, y):
    x2 = x.reshape(_ROWS, _W)
    y2 = y.reshape(_ROWS, _W)
    hxp, hyp = _sc_hist(x2, y2)
    return _tc_loss(hxp, hyp)[0, 0]
